# Initial kernel scaffold; baseline (speedup 1.0000x reference)
import jax
import jax.numpy as jnp
from jax.experimental import pallas as pl


def kernel(x, log_A, Bmat, Cmat, log_delta, skip_D, W_out, b_out):
    raise NotImplementedError("write your pallas kernel here")


# speedup vs baseline: 21.8183x; 21.8183x over previous
import jax
import jax.numpy as jnp
from jax import lax
from jax.experimental import pallas as pl
from jax.experimental.pallas import tpu as pltpu

_EPS = 1e-6
_MIN_DELTA = 1e-3
_T = 128
_GD = 8


def _ssm_kernel(x_ref, la_ref, b_ref, c_ref, ld_ref, skip_ref, y_ref):
    GD, C, B, T = x_ref.shape
    N = la_ref.shape[1]
    CB = C * B
    f32 = jnp.float32

    dt = jax.nn.softplus(ld_ref[...]) + _MIN_DELTA
    A = -jax.nn.softplus(la_ref[...])
    dtA = A * dt
    lam = jnp.exp(dtA)
    dB = (lam - 1.0) / (A + _EPS) * b_ref[...]
    w = c_ref[...] * dB
    lamT = jnp.exp(dtA * float(T))

    i_row = lax.broadcasted_iota(jnp.int32, (T, N), 0).astype(f32)
    i_lane = lax.broadcasted_iota(jnp.int32, (N, T), 1).astype(f32)
    ii = lax.broadcasted_iota(jnp.int32, (T, T), 1)
    jj = lax.broadcasted_iota(jnp.int32, (T, T), 0)
    idx = ii - jj
    causal = idx >= 0
    idx_c = jnp.where(causal, idx, 0)

    for d in range(GD):
        dtA_row = dtA[d:d + 1, :]
        dtA_col = dtA_row.T
        w_col = w[d:d + 1, :].T
        BoT = jnp.exp((float(T - 1) - i_row) * dtA_row)
        AiT = w_col * jnp.exp(dtA_col * (i_lane + 1.0))
        k = jnp.sum(w_col * jnp.exp(dtA_col * i_lane), axis=0, keepdims=True)
        kb = jnp.broadcast_to(k, (T, T))
        ToepT = jnp.where(causal, jnp.take_along_axis(kb, idx_c, axis=1), 0.0)

        Xd = x_ref[d].reshape(CB, T)
        S = jnp.dot(Xd, BoT, preferred_element_type=f32)

        lamT_row = lamT[d:d + 1, :]
        h = jnp.zeros((B, N), f32)
        hs = []
        for c in range(C):
            hs.append(h)
            h = lamT_row * h + S[c * B:(c + 1) * B, :]
        Hprev = jnp.concatenate(hs, axis=0)

        Y = (jnp.dot(Xd, ToepT, preferred_element_type=f32)
             + jnp.dot(Hprev, AiT, preferred_element_type=f32))
        Y = Y + skip_ref[d, 0] * Xd
        y_ref[d] = Y.reshape(C, B, T)


def _proj_kernel(y_ref, w_ref, bias_ref, o_ref):
    B = o_ref.shape[0]
    for b in range(B):
        Z = y_ref[:, 0, b, :]
        o_ref[b, 0] = lax.dot_general(
            Z, w_ref[...], (((0,), (1,)), ((), ())),
            preferred_element_type=jnp.float32) + bias_ref[...]


def kernel(x, log_A, Bmat, Cmat, log_delta, skip_D, W_out, b_out):
    B, L, D = x.shape
    N = log_A.shape[1]
    T = _T
    C = L // T
    GD = _GD

    x4 = x.reshape(B, C, T, D)
    Xt = jnp.transpose(x4, (3, 1, 0, 2))
    ld = log_delta.reshape(D, 1)
    skip = skip_D.reshape(D, 1)

    Yt = pl.pallas_call(
        _ssm_kernel,
        grid=(D // GD,),
        in_specs=[
            pl.BlockSpec((GD, C, B, T), lambda i: (i, 0, 0, 0)),
            pl.BlockSpec((GD, N), lambda i: (i, 0)),
            pl.BlockSpec((GD, N), lambda i: (i, 0)),
            pl.BlockSpec((GD, N), lambda i: (i, 0)),
            pl.BlockSpec((GD, 1), lambda i: (i, 0)),
            pl.BlockSpec((GD, 1), lambda i: (i, 0)),
        ],
        out_specs=pl.BlockSpec((GD, C, B, T), lambda i: (i, 0, 0, 0)),
        out_shape=jax.ShapeDtypeStruct((D, C, B, T), jnp.float32),
        compiler_params=pltpu.CompilerParams(
            dimension_semantics=("parallel",),
        ),
        name="s4_ssm",
    )(Xt, log_A, Bmat, Cmat, ld, skip)

    out4 = pl.pallas_call(
        _proj_kernel,
        grid=(C,),
        in_specs=[
            pl.BlockSpec((D, 1, B, T), lambda c: (0, c, 0, 0)),
            pl.BlockSpec((D, D), lambda c: (0, 0)),
            pl.BlockSpec((1, D), lambda c: (0, 0)),
        ],
        out_specs=pl.BlockSpec((B, 1, T, D), lambda c: (0, c, 0, 0)),
        out_shape=jax.ShapeDtypeStruct((B, C, T, D), jnp.float32),
        compiler_params=pltpu.CompilerParams(
            dimension_semantics=("parallel",),
        ),
        name="s4_proj",
    )(Yt, W_out, b_out.reshape(1, D))

    return out4.reshape(B, L, D)
